# baseline (speedup 1.0000x reference)
import functools

import jax
import jax.numpy as jnp
from jax import lax
from jax.experimental import pallas as pl
from jax.experimental.pallas import tpu as pltpu
from jax.experimental.pallas import tpu_sc as plsc

S_DIM = 1024
T_DIM = 2048
FEAT = 128
N_DATA = 100000
K = 1024
T_NCE = 0.07
EPS = 1e-07
B = 1024

KP = 1040
CHUNK = 80
NCHUNK = KP // CHUNK
GROUPS = CHUNK // 16
NC = 2
NS = 16
NW = NC * NS
B_PER_W = B // NW


def _embed_body(f_s_ref, w_s_ref, b_s_ref, f_t_ref, w_t_ref, b_t_ref,
                v1_ref, v2_ref):
    x1 = jnp.dot(f_s_ref[...], w_s_ref[...].T,
                 preferred_element_type=jnp.float32) + b_s_ref[...]
    n1 = jnp.power(jnp.sum(x1 * x1, axis=1, keepdims=True), 0.5)
    v1_ref[...] = (x1 / n1).astype(jnp.bfloat16)
    x2 = jnp.dot(f_t_ref[...], w_t_ref[...].T,
                 preferred_element_type=jnp.float32) + b_t_ref[...]
    n2 = jnp.power(jnp.sum(x2 * x2, axis=1, keepdims=True), 0.5)
    v2_ref[...] = (x2 / n2).astype(jnp.bfloat16)


def _embed(f_s, W_s, b_s, f_t, W_t, b_t):
    return pl.pallas_call(
        _embed_body,
        out_shape=(jax.ShapeDtypeStruct((B, FEAT), jnp.bfloat16),
                   jax.ShapeDtypeStruct((B, FEAT), jnp.bfloat16)),
    )(f_s, W_s, b_s.reshape(1, FEAT), f_t, W_t, b_t.reshape(1, FEAT))


def _pack16(x):
    n = x.shape[0]
    xb = x.astype(jnp.bfloat16).reshape(n, FEAT // 2, 2)
    return jax.lax.bitcast_convert_type(xb, jnp.int32)


def _sc_body(mem1_hbm, mem2_hbm, cidx_hbm, v1_hbm, v2_hbm,
             e1_hbm, e2_hbm,
             idxb_v, v1b_v, v2b_v,
             r1_0, r1_1, r2_0, r2_1, e1_v, e2_v, e1_w, e2_w,
             s1_0, s1_1, s2_0, s2_1, se1_a, se2_a, se1_b, se2_b):
    wid = lax.axis_index("s") * NC + lax.axis_index("c")
    b0 = wid * B_PER_W
    lane = lax.broadcasted_iota(jnp.int32, (16,), 0)
    inv_t = jnp.float32(1.0 / T_NCE)
    r1 = (r1_0, r1_1)
    r2 = (r2_0, r2_1)
    s1 = (s1_0, s1_1)
    s2 = (s2_0, s2_1)

    pltpu.sync_copy(cidx_hbm.at[pl.ds(b0 * KP, B_PER_W * KP)], idxb_v)
    pltpu.sync_copy(v1_hbm.at[pl.ds(b0, B_PER_W)], v1b_v)
    pltpu.sync_copy(v2_hbm.at[pl.ds(b0, B_PER_W)], v2b_v)

    def fire(i, j, slot):
        isl = idxb_v.at[pl.ds(i * KP + j * CHUNK, CHUNK)]
        pltpu.async_copy(mem1_hbm.at[isl], r1[slot], s1[slot])
        pltpu.async_copy(mem2_hbm.at[isl], r2[slot], s2[slot])

    def wait(slot):
        pltpu.make_async_copy(mem1_hbm.at[idxb_v.at[pl.ds(0, CHUNK)]],
                              r1[slot], s1[slot]).wait()
        pltpu.make_async_copy(mem2_hbm.at[idxb_v.at[pl.ds(0, CHUNK)]],
                              r2[slot], s2[slot]).wait()

    def dots(rows_ref, vv, j, e_ref):
        def g_body(g, _):
            res = jnp.zeros((16,), jnp.float32)
            for pp in range(16):
                row = g * 16 + pp
                p0 = plsc.bitcast(rows_ref[row, pl.ds(0, 16)], jnp.bfloat16) * vv[0]
                p1 = plsc.bitcast(rows_ref[row, pl.ds(16, 16)], jnp.bfloat16) * vv[1]
                p2 = plsc.bitcast(rows_ref[row, pl.ds(32, 16)], jnp.bfloat16) * vv[2]
                p3 = plsc.bitcast(rows_ref[row, pl.ds(48, 16)], jnp.bfloat16) * vv[3]
                s = (p0 + p1) + (p2 + p3)
                ha, hb = plsc.unpack(s, format=plsc.PackFormat.INTERLEAVED)
                res = jnp.where(lane == pp, jnp.sum(ha + hb), res)
            e_ref[pl.ds(j * CHUNK + g * 16, 16)] = jnp.exp(res * inv_t)
            return 0

        lax.fori_loop(0, GROUPS, g_body, 0)

    def do_chunk(i, j, slot, vv1, vv2, e1_ref, e2_ref):
        wait(slot)
        dots(r2[slot], vv1, j, e1_ref)
        dots(r1[slot], vv2, j, e2_ref)

    fire(0, 0, 0)
    fire(0, 1, 1)

    def half(i, guard, e1_ref, e2_ref, se1, se2):
        @pl.when(guard)
        def _():
            pltpu.make_async_copy(e1_ref, e1_hbm.at[b0], se1).wait()
            pltpu.make_async_copy(e2_ref, e2_hbm.at[b0], se2).wait()

        vv1 = [plsc.bitcast(v1b_v[i, pl.ds(fc * 16, 16)], jnp.bfloat16)
               for fc in range(4)]
        vv2 = [plsc.bitcast(v2b_v[i, pl.ds(fc * 16, 16)], jnp.bfloat16)
               for fc in range(4)]

        def jj_body(jj, _):
            j0 = 2 * jj
            do_chunk(i, j0, 0, vv1, vv2, e1_ref, e2_ref)
            fire(i, j0 + 2, 0)
            do_chunk(i, j0 + 1, 1, vv1, vv2, e1_ref, e2_ref)

            @pl.when(jj < (NCHUNK - 3) // 2)
            def _():
                fire(i, j0 + 3, 1)

            @pl.when((jj == (NCHUNK - 3) // 2) & (i < B_PER_W - 1))
            def _():
                fire(i + 1, 1, 1)
            return 0

        lax.fori_loop(0, (NCHUNK - 1) // 2, jj_body, 0)

        do_chunk(i, NCHUNK - 1, 0, vv1, vv2, e1_ref, e2_ref)

        @pl.when(i < B_PER_W - 1)
        def _():
            fire(i + 1, 0, 0)

        pltpu.async_copy(e1_ref, e1_hbm.at[b0 + i], se1)
        pltpu.async_copy(e2_ref, e2_hbm.at[b0 + i], se2)

    def b_body(ii, _):
        half(2 * ii, ii > 0, e1_v, e2_v, se1_a, se2_a)
        half(2 * ii + 1, ii > 0, e1_w, e2_w, se1_b, se2_b)
        return 0

    lax.fori_loop(0, B_PER_W // 2, b_body, 0)

    pltpu.make_async_copy(e1_v, e1_hbm.at[b0], se1_a).wait()
    pltpu.make_async_copy(e2_v, e2_hbm.at[b0], se2_a).wait()
    pltpu.make_async_copy(e1_w, e1_hbm.at[b0], se1_b).wait()
    pltpu.make_async_copy(e2_w, e2_hbm.at[b0], se2_b).wait()


def _sc_scores(memory_v1, memory_v2, cidx_pad, v1, v2):
    mesh = plsc.VectorSubcoreMesh(core_axis_name="c", subcore_axis_name="s",
                                  num_cores=NC, num_subcores=NS)
    f = pl.kernel(
        _sc_body,
        out_type=(jax.ShapeDtypeStruct((B, KP), jnp.float32),
                  jax.ShapeDtypeStruct((B, KP), jnp.float32)),
        mesh=mesh,
        compiler_params=pltpu.CompilerParams(needs_layout_passes=False,
                                             use_tc_tiling_on_sc=False),
        scratch_types=[
            pltpu.VMEM((B_PER_W * KP,), jnp.int32),
            pltpu.VMEM((B_PER_W, FEAT // 2), jnp.int32),
            pltpu.VMEM((B_PER_W, FEAT // 2), jnp.int32),
            pltpu.VMEM((CHUNK, FEAT // 2), jnp.int32),
            pltpu.VMEM((CHUNK, FEAT // 2), jnp.int32),
            pltpu.VMEM((CHUNK, FEAT // 2), jnp.int32),
            pltpu.VMEM((CHUNK, FEAT // 2), jnp.int32),
            pltpu.VMEM((KP,), jnp.float32),
            pltpu.VMEM((KP,), jnp.float32),
            pltpu.VMEM((KP,), jnp.float32),
            pltpu.VMEM((KP,), jnp.float32),
            pltpu.SemaphoreType.DMA,
            pltpu.SemaphoreType.DMA,
            pltpu.SemaphoreType.DMA,
            pltpu.SemaphoreType.DMA,
            pltpu.SemaphoreType.DMA,
            pltpu.SemaphoreType.DMA,
            pltpu.SemaphoreType.DMA,
            pltpu.SemaphoreType.DMA,
        ],
    )
    return f(memory_v1, memory_v2, cidx_pad, v1, v2)


def _loss_body(e1_ref, e2_ref, out_ref):
    m = jnp.float32(K)
    pn = jnp.float32(1.0 / N_DATA)
    col = lax.broadcasted_iota(jnp.int32, (B, KP), 1)
    valid = (col < K + 1).astype(jnp.float32)
    neg = ((col >= 1) & (col < K + 1)).astype(jnp.float32)

    def one(e):
        z = jnp.sum(e * valid) / jnp.float32(B * (K + 1)) * jnp.float32(N_DATA)
        o = e / z
        p_pos = o[:, 0]
        ld1 = jnp.sum(jnp.log(p_pos / (p_pos + m * pn + EPS)))
        ld0 = jnp.sum(jnp.log((m * pn) / (o + m * pn + EPS)) * neg)
        return -(ld1 + ld0) / jnp.float32(B)

    out_ref[0, 0] = one(e1_ref[...]) + one(e2_ref[...])


def _loss(e1, e2):
    out = pl.pallas_call(
        _loss_body,
        out_shape=jax.ShapeDtypeStruct((1, 1), jnp.float32),
        out_specs=pl.BlockSpec(memory_space=pltpu.SMEM),
    )(e1, e2)
    return out.reshape(1)


def kernel(f_s, f_t, idx, contrast_idx, W_s, b_s, W_t, b_t,
           memory_v1, memory_v2):
    del idx
    v1, v2 = _embed(f_s, W_s, b_s, f_t, W_t, b_t)
    v1 = jax.lax.bitcast_convert_type(v1.reshape(B, FEAT // 2, 2), jnp.int32)
    v2 = jax.lax.bitcast_convert_type(v2.reshape(B, FEAT // 2, 2), jnp.int32)
    cidx_pad = jnp.pad(contrast_idx.astype(jnp.int32),
                       ((0, 0), (0, KP - (K + 1))))
    m1 = _pack16(memory_v1)
    m2 = _pack16(memory_v2)
    e1, e2 = _sc_scores(m1, m2, cidx_pad.reshape(-1), v1, v2)
    return _loss(e1, e2)


# speedup vs baseline: 1.9382x; 1.9382x over previous
import functools

import jax
import jax.numpy as jnp
from jax import lax
from jax.experimental import pallas as pl
from jax.experimental.pallas import tpu as pltpu
from jax.experimental.pallas import tpu_sc as plsc

S_DIM = 1024
T_DIM = 2048
FEAT = 128
N_DATA = 100000
K = 1024
T_NCE = 0.07
EPS = 1e-07
B = 1024

KP = 1040
CHUNK = 80
NCHUNK = KP // CHUNK
GROUPS = CHUNK // 16
NC = 2
NS = 16
NW = NC * NS
B_PER_W = B // NW


def _embed_body(f_s_ref, w_s_ref, b_s_ref, f_t_ref, w_t_ref, b_t_ref,
                v1_ref, v2_ref):
    x1 = jnp.dot(f_s_ref[...], w_s_ref[...].T,
                 preferred_element_type=jnp.float32) + b_s_ref[...]
    n1 = jnp.power(jnp.sum(x1 * x1, axis=1, keepdims=True), 0.5)
    v1_ref[...] = _pack_halves(x1 / n1)
    x2 = jnp.dot(f_t_ref[...], w_t_ref[...].T,
                 preferred_element_type=jnp.float32) + b_t_ref[...]
    n2 = jnp.power(jnp.sum(x2 * x2, axis=1, keepdims=True), 0.5)
    v2_ref[...] = _pack_halves(x2 / n2)


def _embed(f_s, W_s, b_s, f_t, W_t, b_t):
    return pl.pallas_call(
        _embed_body,
        out_shape=(jax.ShapeDtypeStruct((B, FEAT // 2), jnp.int32),
                   jax.ShapeDtypeStruct((B, FEAT // 2), jnp.int32)),
    )(f_s, W_s, b_s.reshape(1, FEAT), f_t, W_t, b_t.reshape(1, FEAT))


def _rne_bf16_bits(x):
    u = jax.lax.bitcast_convert_type(x, jnp.uint32)
    r = u + jnp.uint32(0x7FFF) + ((u >> 16) & jnp.uint32(1))
    return r >> 16


def _pack_halves(x):
    t = _rne_bf16_bits(x)
    lo = t[:, : FEAT // 2]
    hi = t[:, FEAT // 2:]
    return jax.lax.bitcast_convert_type(lo | (hi << 16), jnp.int32)


def _pack_body(x_ref, o_ref):
    o_ref[...] = _pack_halves(x_ref[...])


def _pack16(x):
    n = x.shape[0]
    blk = 2000
    return pl.pallas_call(
        _pack_body,
        grid=(n // blk,),
        in_specs=[pl.BlockSpec((blk, FEAT), lambda i: (i, 0))],
        out_specs=pl.BlockSpec((blk, FEAT // 2), lambda i: (i, 0)),
        out_shape=jax.ShapeDtypeStruct((n, FEAT // 2), jnp.int32),
    )(x)


def _sc_body(mem1_hbm, mem2_hbm, cidx_hbm, v1_hbm, v2_hbm,
             e1_hbm, e2_hbm,
             idxb_v, v1b_v, v2b_v,
             r1_0, r1_1, r2_0, r2_1, e1_v, e2_v, e1_w, e2_w,
             s1_0, s1_1, s2_0, s2_1, se1_a, se2_a, se1_b, se2_b):
    wid = lax.axis_index("s") * NC + lax.axis_index("c")
    b0 = wid * B_PER_W
    lane = lax.broadcasted_iota(jnp.int32, (16,), 0)
    inv_t = jnp.float32(1.0 / T_NCE)
    r1 = (r1_0, r1_1)
    r2 = (r2_0, r2_1)
    s1 = (s1_0, s1_1)
    s2 = (s2_0, s2_1)

    pltpu.sync_copy(cidx_hbm.at[pl.ds(b0 * KP, B_PER_W * KP)], idxb_v)
    pltpu.sync_copy(v1_hbm.at[pl.ds(b0, B_PER_W)], v1b_v)
    pltpu.sync_copy(v2_hbm.at[pl.ds(b0, B_PER_W)], v2b_v)

    def fire(i, j, slot):
        isl = idxb_v.at[pl.ds(i * KP + j * CHUNK, CHUNK)]
        pltpu.async_copy(mem1_hbm.at[isl], r1[slot], s1[slot])
        pltpu.async_copy(mem2_hbm.at[isl], r2[slot], s2[slot])

    def wait(slot):
        pltpu.make_async_copy(mem1_hbm.at[idxb_v.at[pl.ds(0, CHUNK)]],
                              r1[slot], s1[slot]).wait()
        pltpu.make_async_copy(mem2_hbm.at[idxb_v.at[pl.ds(0, CHUNK)]],
                              r2[slot], s2[slot]).wait()

    def dots(rows_ref, vv, j, e_ref):
        def g_body(g, _):
            res = jnp.zeros((16,), jnp.float32)
            for pp in range(16):
                row = g * 16 + pp
                p0 = plsc.bitcast(rows_ref[row, pl.ds(0, 16)], jnp.bfloat16) * vv[0]
                p1 = plsc.bitcast(rows_ref[row, pl.ds(16, 16)], jnp.bfloat16) * vv[1]
                p2 = plsc.bitcast(rows_ref[row, pl.ds(32, 16)], jnp.bfloat16) * vv[2]
                p3 = plsc.bitcast(rows_ref[row, pl.ds(48, 16)], jnp.bfloat16) * vv[3]
                s = (p0 + p1) + (p2 + p3)
                ha, hb = plsc.unpack(s, format=plsc.PackFormat.INTERLEAVED)
                res = jnp.where(lane == pp, jnp.sum(ha + hb), res)
            e_ref[pl.ds(j * CHUNK + g * 16, 16)] = jnp.exp(res * inv_t)
            return 0

        lax.fori_loop(0, GROUPS, g_body, 0)

    def do_chunk(i, j, slot, vv1, vv2, e1_ref, e2_ref):
        wait(slot)
        dots(r2[slot], vv1, j, e1_ref)
        dots(r1[slot], vv2, j, e2_ref)

    fire(0, 0, 0)
    fire(0, 1, 1)

    def half(i, guard, e1_ref, e2_ref, se1, se2):
        @pl.when(guard)
        def _():
            pltpu.make_async_copy(e1_ref, e1_hbm.at[b0], se1).wait()
            pltpu.make_async_copy(e2_ref, e2_hbm.at[b0], se2).wait()

        vv1 = [plsc.bitcast(v1b_v[i, pl.ds(fc * 16, 16)], jnp.bfloat16)
               for fc in range(4)]
        vv2 = [plsc.bitcast(v2b_v[i, pl.ds(fc * 16, 16)], jnp.bfloat16)
               for fc in range(4)]

        def jj_body(jj, _):
            j0 = 2 * jj
            do_chunk(i, j0, 0, vv1, vv2, e1_ref, e2_ref)
            fire(i, j0 + 2, 0)
            do_chunk(i, j0 + 1, 1, vv1, vv2, e1_ref, e2_ref)

            @pl.when(jj < (NCHUNK - 3) // 2)
            def _():
                fire(i, j0 + 3, 1)

            @pl.when((jj == (NCHUNK - 3) // 2) & (i < B_PER_W - 1))
            def _():
                fire(i + 1, 1, 1)
            return 0

        lax.fori_loop(0, (NCHUNK - 1) // 2, jj_body, 0)

        do_chunk(i, NCHUNK - 1, 0, vv1, vv2, e1_ref, e2_ref)

        @pl.when(i < B_PER_W - 1)
        def _():
            fire(i + 1, 0, 0)

        pltpu.async_copy(e1_ref, e1_hbm.at[b0 + i], se1)
        pltpu.async_copy(e2_ref, e2_hbm.at[b0 + i], se2)

    def b_body(ii, _):
        half(2 * ii, ii > 0, e1_v, e2_v, se1_a, se2_a)
        half(2 * ii + 1, ii > 0, e1_w, e2_w, se1_b, se2_b)
        return 0

    lax.fori_loop(0, B_PER_W // 2, b_body, 0)

    pltpu.make_async_copy(e1_v, e1_hbm.at[b0], se1_a).wait()
    pltpu.make_async_copy(e2_v, e2_hbm.at[b0], se2_a).wait()
    pltpu.make_async_copy(e1_w, e1_hbm.at[b0], se1_b).wait()
    pltpu.make_async_copy(e2_w, e2_hbm.at[b0], se2_b).wait()


def _sc_scores(memory_v1, memory_v2, cidx_pad, v1, v2):
    mesh = plsc.VectorSubcoreMesh(core_axis_name="c", subcore_axis_name="s",
                                  num_cores=NC, num_subcores=NS)
    f = pl.kernel(
        _sc_body,
        out_type=(jax.ShapeDtypeStruct((B, KP), jnp.float32),
                  jax.ShapeDtypeStruct((B, KP), jnp.float32)),
        mesh=mesh,
        compiler_params=pltpu.CompilerParams(needs_layout_passes=False,
                                             use_tc_tiling_on_sc=False),
        scratch_types=[
            pltpu.VMEM((B_PER_W * KP,), jnp.int32),
            pltpu.VMEM((B_PER_W, FEAT // 2), jnp.int32),
            pltpu.VMEM((B_PER_W, FEAT // 2), jnp.int32),
            pltpu.VMEM((CHUNK, FEAT // 2), jnp.int32),
            pltpu.VMEM((CHUNK, FEAT // 2), jnp.int32),
            pltpu.VMEM((CHUNK, FEAT // 2), jnp.int32),
            pltpu.VMEM((CHUNK, FEAT // 2), jnp.int32),
            pltpu.VMEM((KP,), jnp.float32),
            pltpu.VMEM((KP,), jnp.float32),
            pltpu.VMEM((KP,), jnp.float32),
            pltpu.VMEM((KP,), jnp.float32),
            pltpu.SemaphoreType.DMA,
            pltpu.SemaphoreType.DMA,
            pltpu.SemaphoreType.DMA,
            pltpu.SemaphoreType.DMA,
            pltpu.SemaphoreType.DMA,
            pltpu.SemaphoreType.DMA,
            pltpu.SemaphoreType.DMA,
            pltpu.SemaphoreType.DMA,
        ],
    )
    return f(memory_v1, memory_v2, cidx_pad, v1, v2)


def _loss_body(e1_ref, e2_ref, out_ref):
    m = jnp.float32(K)
    pn = jnp.float32(1.0 / N_DATA)
    col = lax.broadcasted_iota(jnp.int32, (B, KP), 1)
    valid = (col < K + 1).astype(jnp.float32)
    neg = ((col >= 1) & (col < K + 1)).astype(jnp.float32)

    def one(e):
        z = jnp.sum(e * valid) / jnp.float32(B * (K + 1)) * jnp.float32(N_DATA)
        o = e / z
        p_pos = o[:, 0]
        ld1 = jnp.sum(jnp.log(p_pos / (p_pos + m * pn + EPS)))
        ld0 = jnp.sum(jnp.log((m * pn) / (o + m * pn + EPS)) * neg)
        return -(ld1 + ld0) / jnp.float32(B)

    out_ref[0, 0] = one(e1_ref[...]) + one(e2_ref[...])


def _loss(e1, e2):
    out = pl.pallas_call(
        _loss_body,
        out_shape=jax.ShapeDtypeStruct((1, 1), jnp.float32),
        out_specs=pl.BlockSpec(memory_space=pltpu.SMEM),
    )(e1, e2)
    return out.reshape(1)


def kernel(f_s, f_t, idx, contrast_idx, W_s, b_s, W_t, b_t,
           memory_v1, memory_v2):
    del idx
    v1, v2 = _embed(f_s, W_s, b_s, f_t, W_t, b_t)
    cidx_pad = jnp.pad(contrast_idx.astype(jnp.int32),
                       ((0, 0), (0, KP - (K + 1))))
    m1 = _pack16(memory_v1)
    m2 = _pack16(memory_v2)
    e1, e2 = _sc_scores(m1, m2, cidx_pad.reshape(-1), v1, v2)
    return _loss(e1, e2)
